# Initial kernel scaffold; baseline (speedup 1.0000x reference)
import jax
import jax.numpy as jnp
from jax.experimental import pallas as pl


def kernel(x, edge_index, edge_attr, params):
    raise NotImplementedError("write your pallas kernel here")


# speedup vs baseline: 52.8158x; 52.8158x over previous
import functools
import math

import jax
import jax.numpy as jnp
from jax import lax
from jax.experimental import pallas as pl
from jax.experimental.pallas import tpu as pltpu
from jax.experimental.pallas import tpu_sc as plsc

N = 10000
E = 320000
D = 128
H = 8
C = 16
NW = 32
CHUNK = 80
NCHUNKS = E // CHUNK
WCHUNKS = NCHUNKS // NW
NPAD = 10240
ROWS_PER_TILE = NPAD // 16
ZB = 128
NB = 10
BN = N // NB
EB = 100
BE = E // EB
MROWS = BE // CHUNK


def _r2(b):
    return b.reshape(1, -1)


def _node_in_body(x_ref, w0, b0, w1, b1, w2, b2, o_ref):
    h = jnp.maximum(jnp.dot(x_ref[...], w0[...], preferred_element_type=jnp.float32) + b0[...], 0.0)
    h = jnp.maximum(jnp.dot(h, w1[...], preferred_element_type=jnp.float32) + b1[...], 0.0)
    o_ref[...] = jnp.dot(h, w2[...], preferred_element_type=jnp.float32) + b2[...]


def _node_in(x, p):
    return pl.pallas_call(
        _node_in_body,
        grid=(NB,),
        in_specs=[
            pl.BlockSpec((BN, 128), lambda i: (i, 0)),
            pl.BlockSpec((128, D), lambda i: (0, 0)),
            pl.BlockSpec((1, D), lambda i: (0, 0)),
            pl.BlockSpec((D, D), lambda i: (0, 0)),
            pl.BlockSpec((1, D), lambda i: (0, 0)),
            pl.BlockSpec((D, D), lambda i: (0, 0)),
            pl.BlockSpec((1, D), lambda i: (0, 0)),
        ],
        out_specs=pl.BlockSpec((BN, D), lambda i: (i, 0)),
        out_shape=jax.ShapeDtypeStruct((N, D), jnp.float32),
    )(x, p["w0"], _r2(p["b0"]), p["w1"], _r2(p["b1"]), p["w2"], _r2(p["b2"]))


def _edge_pre_body(ea_ref, w0, b0, w1, b1, w2, b2,
                   we0, be0, we1, be1, we2, be2,
                   e0_ref, e1_ref, e2_ref, m_ref):
    a = ea_ref[:, :14]
    h = jnp.maximum(jnp.dot(a, w0[...], preferred_element_type=jnp.float32) + b0[...], 0.0)
    h = jnp.maximum(jnp.dot(h, w1[...], preferred_element_type=jnp.float32) + b1[...], 0.0)
    emb = jnp.dot(h, w2[...], preferred_element_type=jnp.float32) + b2[...]
    for we, be, ref in ((we0, be0, e0_ref), (we1, be1, e1_ref), (we2, be2, e2_ref)):
        ref[...] = jnp.dot(emb, we[...], preferred_element_type=jnp.float32) + be[...]
    m = ea_ref[:, 14] * (1.0 - ea_ref[:, 15]) * (0.25)
    m2 = m.reshape(MROWS, CHUNK)
    m_ref[...] = jnp.concatenate(
        [m2, jnp.zeros((MROWS, 128 - CHUNK), jnp.float32)], axis=1)


def _edge_pre(edge_attr, pe, convs):
    wspec = lambda shp: pl.BlockSpec(shp, lambda i: (0, 0))
    return pl.pallas_call(
        _edge_pre_body,
        grid=(EB,),
        in_specs=[
            pl.BlockSpec((BE, 16), lambda i: (i, 0)),
            wspec((14, 16)), wspec((1, 16)),
            wspec((16, 16)), wspec((1, 16)),
            wspec((16, 16)), wspec((1, 16)),
            wspec((16, D)), wspec((1, D)),
            wspec((16, D)), wspec((1, D)),
            wspec((16, D)), wspec((1, D)),
        ],
        out_specs=[
            pl.BlockSpec((BE, D), lambda i: (i, 0)),
            pl.BlockSpec((BE, D), lambda i: (i, 0)),
            pl.BlockSpec((BE, D), lambda i: (i, 0)),
            pl.BlockSpec((MROWS, 128), lambda i: (i, 0)),
        ],
        out_shape=[
            jax.ShapeDtypeStruct((E, D), jnp.float32),
            jax.ShapeDtypeStruct((E, D), jnp.float32),
            jax.ShapeDtypeStruct((E, D), jnp.float32),
            jax.ShapeDtypeStruct((NCHUNKS, 128), jnp.float32),
        ],
    )(edge_attr, pe["w0"], _r2(pe["b0"]), pe["w1"], _r2(pe["b1"]),
      pe["w2"], _r2(pe["b2"]),
      convs[0]["we"], _r2(convs[0]["be"]),
      convs[1]["we"], _r2(convs[1]["be"]),
      convs[2]["we"], _r2(convs[2]["be"]))


def _qkv_first_body(x_ref, wq, bq, wk, bk, wv, bv, ws, bs,
                    q_ref, k_ref, v_ref, s_ref):
    x = x_ref[...]
    q_ref[...] = jnp.dot(x, wq[...], preferred_element_type=jnp.float32) + bq[...]
    k_ref[...] = jnp.dot(x, wk[...], preferred_element_type=jnp.float32) + bk[...]
    v_ref[...] = jnp.dot(x, wv[...], preferred_element_type=jnp.float32) + bv[...]
    s_ref[...] = jnp.dot(x, ws[...], preferred_element_type=jnp.float32) + bs[...]


def _qkv_next_body(a_ref, b_ref, sk_ref, wq, bq, wk, bk, wv, bv, ws, bs,
                   q_ref, k_ref, v_ref, s_ref):
    x = a_ref[...] + b_ref[...] + sk_ref[...]
    q_ref[...] = jnp.dot(x, wq[...], preferred_element_type=jnp.float32) + bq[...]
    k_ref[...] = jnp.dot(x, wk[...], preferred_element_type=jnp.float32) + bk[...]
    v_ref[...] = jnp.dot(x, wv[...], preferred_element_type=jnp.float32) + bv[...]
    s_ref[...] = jnp.dot(x, ws[...], preferred_element_type=jnp.float32) + bs[...]


def _qkv_specs(nin):
    wspec = lambda: pl.BlockSpec((D, D), lambda i: (0, 0))
    bspec = lambda: pl.BlockSpec((1, D), lambda i: (0, 0))
    xspec = [pl.BlockSpec((BN, D), lambda i: (i, 0)) for _ in range(nin)]
    specs = xspec + [s for _ in range(4) for s in (wspec(), bspec())]
    return dict(
        grid=(NB,),
        in_specs=specs,
        out_specs=[pl.BlockSpec((BN, D), lambda i: (i, 0)) for _ in range(4)],
        out_shape=[jax.ShapeDtypeStruct((N, D), jnp.float32)] * 4,
    )


def _qkv_first(x, p):
    return pl.pallas_call(_qkv_first_body, **_qkv_specs(1))(
        x, p["wq"], _r2(p["bq"]), p["wk"], _r2(p["bk"]),
        p["wv"], _r2(p["bv"]), p["ws"], _r2(p["bs"]))


def _qkv_next(a, b, sk, p):
    return pl.pallas_call(_qkv_next_body, **_qkv_specs(3))(
        a, b, sk, p["wq"], _r2(p["bq"]), p["wk"], _r2(p["bk"]),
        p["wv"], _r2(p["bv"]), p["ws"], _r2(p["bs"]))


def _out_mlp_body(a_ref, b_ref, sk_ref, w0, b0, w1, b1, w2, b2, o_ref):
    x = a_ref[...] + b_ref[...] + sk_ref[...]
    h = jnp.maximum(jnp.dot(x, w0[...], preferred_element_type=jnp.float32) + b0[...], 0.0)
    h = jnp.maximum(jnp.dot(h, w1[...], preferred_element_type=jnp.float32) + b1[...], 0.0)
    o_ref[...] = jax.nn.sigmoid(jnp.dot(h, w2[...], preferred_element_type=jnp.float32) + b2[...])


def _out_mlp(a, b, sk, p):
    wspec = lambda shp: pl.BlockSpec(shp, lambda i: (0, 0))
    return pl.pallas_call(
        _out_mlp_body,
        grid=(NB,),
        in_specs=[
            pl.BlockSpec((BN, D), lambda i: (i, 0)),
            pl.BlockSpec((BN, D), lambda i: (i, 0)),
            pl.BlockSpec((BN, D), lambda i: (i, 0)),
            wspec((D, D)), wspec((1, D)),
            wspec((D, D)), wspec((1, D)),
            wspec((D, 1)), wspec((1, 1)),
        ],
        out_specs=pl.BlockSpec((BN, 1), lambda i: (i, 0)),
        out_shape=jax.ShapeDtypeStruct((N, 1), jnp.float32),
    )(a, b, sk, p["w0"], _r2(p["b0"]), p["w1"], _r2(p["b1"]),
      p["w2"], _r2(p["b2"]))


_SC_MESH = plsc.VectorSubcoreMesh(core_axis_name="c", subcore_axis_name="s")


def _sc_edge_body(q_hbm, k_hbm, v_hbm, ea_hbm, m_hbm, src_hbm, dst_hbm,
                  out_hbm,
                  src_v, dst_v, m_v, q_rows, k_rows, v_rows, ea_rows,
                  acc, s0, s1, s2, s3):
    core = lax.axis_index("c")
    sub = lax.axis_index("s")
    wid = sub * 2 + core

    lanes = lax.iota(jnp.int32, 16)
    perms = [(lanes ^ 8)[:, None], (lanes ^ 4)[:, None],
             (lanes ^ 2)[:, None], (lanes ^ 1)[:, None]]
    zero16 = (lanes & 0)[:, None]
    _dn = lax.GatherDimensionNumbers(
        offset_dims=(), collapsed_slice_dims=(0,), start_index_map=(0,))

    def bcast0(xv):
        return lax.gather(xv, zero16, _dn, slice_sizes=(1,),
                          mode=lax.GatherScatterMode.PROMISE_IN_BOUNDS)

    def vsum(xv):
        for p in perms:
            xv = xv + lax.gather(
                xv, p, _dn, slice_sizes=(1,),
                mode=lax.GatherScatterMode.PROMISE_IN_BOUNDS)
        return xv

    zz = jnp.zeros((16,), jnp.float32)

    def zrow(r, carry):
        for j in range(D // 16):
            v_rows[r, pl.ds(j * 16, 16)] = zz
        return carry

    lax.fori_loop(0, CHUNK, zrow, 0)
    for t in range(ROWS_PER_TILE // CHUNK):
        pltpu.sync_copy(v_rows,
                        acc.at[pl.ds(sub * ROWS_PER_TILE + t * CHUNK, CHUNK)])
    plsc.subcore_barrier()

    def chunk_body(t, carry):
        cidx = wid * WCHUNKS + t
        base = cidx * CHUNK
        pltpu.sync_copy(src_hbm.at[pl.ds(base, CHUNK)], src_v)
        pltpu.sync_copy(dst_hbm.at[pl.ds(base, CHUNK)], dst_v)
        cq = pltpu.async_copy(q_hbm.at[dst_v], q_rows, s0)
        ck = pltpu.async_copy(k_hbm.at[src_v], k_rows, s1)
        cv = pltpu.async_copy(v_hbm.at[src_v], v_rows, s2)
        ce = pltpu.async_copy(ea_hbm.at[pl.ds(base, CHUNK)], ea_rows, s3)
        pltpu.sync_copy(m_hbm.at[pl.ds(cidx, 1)], m_v)
        cq.wait()
        ck.wait()
        cv.wait()
        ce.wait()

        def edge_body(e, ecarry):
            m = bcast0(m_v[0, pl.ds(e, 16)])
            for h in range(H):
                sl = pl.ds(h * C, C)
                qv = q_rows[e, sl]
                kv = k_rows[e, sl]
                ev = ea_rows[e, sl]
                s = vsum(qv * (kv + ev))
                v_rows[e, sl] = (v_rows[e, sl] + ev) * (s * m)
            return ecarry

        lax.fori_loop(0, CHUNK, edge_body, 0)
        pltpu.sync_copy(v_rows, acc.at[dst_v], add=True)
        return carry

    lax.fori_loop(0, WCHUNKS, chunk_body, 0)
    plsc.subcore_barrier()

    for t in range(ROWS_PER_TILE // ZB):
        r = sub * ROWS_PER_TILE + t * ZB
        pltpu.sync_copy(acc.at[pl.ds(r, ZB)],
                        out_hbm.at[pl.ds(core * NPAD + r, ZB)])


_sc_edge = pl.kernel(
    _sc_edge_body,
    out_type=jax.ShapeDtypeStruct((2 * NPAD, D), jnp.float32),
    mesh=_SC_MESH,
    scratch_types=[
        pltpu.VMEM((CHUNK,), jnp.int32),
        pltpu.VMEM((CHUNK,), jnp.int32),
        pltpu.VMEM((1, 128), jnp.float32),
        pltpu.VMEM((CHUNK, D), jnp.float32),
        pltpu.VMEM((CHUNK, D), jnp.float32),
        pltpu.VMEM((CHUNK, D), jnp.float32),
        pltpu.VMEM((CHUNK, D), jnp.float32),
        pltpu.VMEM_SHARED((NPAD, D), jnp.float32),
        pltpu.SemaphoreType.DMA,
        pltpu.SemaphoreType.DMA,
        pltpu.SemaphoreType.DMA,
        pltpu.SemaphoreType.DMA,
    ],
)


def kernel(x, edge_index, edge_attr, params):
    src = edge_index[0]
    dst = edge_index[1]

    x1 = _node_in(x, params["i_lin"])
    ea0, ea1, ea2, m2 = _edge_pre(edge_attr, params["edge_lin"],
                                  params["convs"])
    eas = (ea0, ea1, ea2)

    q, k, v, sk = _qkv_first(x1, params["convs"][0])
    out = None
    for l in range(3):
        agg = _sc_edge(q, k, v, eas[l], m2, src, dst)
        a_lo = agg[:N]
        a_hi = agg[NPAD:NPAD + N]
        if l < 2:
            q, k, v, sk = _qkv_next(a_lo, a_hi, sk, params["convs"][l + 1])
        else:
            out = _out_mlp(a_lo, a_hi, sk, params["o_lin"])
    return out
